# Initial kernel scaffold; baseline (speedup 1.0000x reference)
import jax
import jax.numpy as jnp
from jax.experimental import pallas as pl


def kernel(x, edge_index, edge_attr, W0, b0, g0, be0, W1, b1, g1, be1, W2, b2, g2, be2, W3, b3, g3, be3, fc1_W, fc1_b, fc2_W, fc2_b):
    raise NotImplementedError("write your pallas kernel here")


# speedup vs baseline: 2.2210x; 2.2210x over previous
import functools

import jax
import jax.numpy as jnp
from jax import lax
from jax.experimental import pallas as pl
from jax.experimental.pallas import tpu as pltpu
from jax.experimental.pallas import tpu_sc as plsc

N = 10000
NP = 10240
E = 160000
D_IN = 256
H = 512
NSC = 2
NTEC = 16
EPT = E // NTEC
K = 80
NBLK = EPT // K
STRIPE = NP // NTEC
EPS = 1e-5

_f32 = jnp.float32
_i32 = jnp.int32
_bf16 = jnp.bfloat16

_GDN = lax.GatherDimensionNumbers(
    offset_dims=(), collapsed_slice_dims=(0,), start_index_map=(0,))


def _lane_bcast(vec, lane):
    idx = jnp.full((16, 1), lane, _i32)
    return lax.gather(vec, idx, dimension_numbers=_GDN, slice_sizes=(1,),
                      mode=lax.GatherScatterMode.PROMISE_IN_BOUNDS)


def _sc_agg_body(nq, *refs):
    (x2d, col, row, ew, aggout,
     acc, colbuf, rowbuf, gidxbuf, ewbuf, rowsbuf,
     sem) = refs
    qper = nq // NSC
    c = lax.axis_index("c")
    s = lax.axis_index("s")
    tbase = s * STRIPE

    zero16 = jnp.zeros((16,), _f32)

    def _zrow(i, carry):
        rr = rowsbuf.at[i]
        for j in range(8):
            rr[pl.ds(j * 16, 16)] = zero16
        return carry
    lax.fori_loop(0, K, _zrow, 0)

    for qi in range(qper):
        q = c * qper + qi

        for z in range(STRIPE // K):
            pltpu.sync_copy(rowsbuf, acc.at[pl.ds(tbase + z * K, K)])
        plsc.subcore_barrier()

        def _block(b, carry):
            base = s * EPT + b * K
            pltpu.sync_copy(col.at[pl.ds(base, K)], colbuf)
            pltpu.sync_copy(row.at[pl.ds(base, K)], rowbuf)
            pltpu.sync_copy(ew.at[pl.ds(base, K)], ewbuf)
            for jj in range(K // 16):
                cv = colbuf[pl.ds(jj * 16, 16)]
                gidxbuf[pl.ds(jj * 16, 16)] = cv * nq + q
            pltpu.async_copy(x2d.at[gidxbuf], rowsbuf, sem).wait()

            def _scale(grp, inner):
                ewv = ewbuf[pl.ds(grp * 16, 16)]
                for lane in range(16):
                    w = _lane_bcast(ewv, lane)
                    rr = rowsbuf.at[grp * 16 + lane]
                    for j in range(8):
                        sl = pl.ds(j * 16, 16)
                        rr[sl] = rr[sl] * w
                return inner
            lax.fori_loop(0, K // 16, _scale, 0)

            pltpu.sync_copy(rowsbuf, acc.at[rowbuf], add=True)
            return carry
        lax.fori_loop(0, NBLK, _block, 0)
        plsc.subcore_barrier()

        pltpu.sync_copy(acc.at[pl.ds(tbase, STRIPE)],
                        aggout.at[pl.ds(tbase, STRIPE), pl.ds(q * 128, 128)])
        plsc.subcore_barrier()


def _make_sc_agg(din):
    nq = din // 128
    mesh = plsc.VectorSubcoreMesh(core_axis_name="c", subcore_axis_name="s")
    scratch = [
        pltpu.VMEM_SHARED((NP, 128), _f32),
        pltpu.VMEM((K,), _i32),
        pltpu.VMEM((K,), _i32),
        pltpu.VMEM((K,), _i32),
        pltpu.VMEM((K,), _f32),
        pltpu.VMEM((K, 128), _f32),
        pltpu.SemaphoreType.DMA,
    ]

    def body(x2d, col, row, ew, *rest):
        _sc_agg_body(nq, x2d, col, row, ew, *rest)

    return pl.kernel(body, out_type=jax.ShapeDtypeStruct((NP, din), _f32),
                     mesh=mesh, scratch_types=scratch)


def _deg_body(row, ew, degout, degacc, rowbuf, ewbuf, degsrc):
    c = lax.axis_index("c")
    s = lax.axis_index("s")
    tbase = s * STRIPE
    zero16 = jnp.zeros((16,), _f32)

    def _zdeg(i, carry):
        dd = degsrc.at[i]
        for j in range(8):
            dd[pl.ds(j * 16, 16)] = zero16
        return carry
    lax.fori_loop(0, K, _zdeg, 0)
    for z in range(STRIPE // K):
        pltpu.sync_copy(degsrc, degacc.at[pl.ds(tbase + z * K, K)])
    plsc.subcore_barrier()

    @pl.when(c == 0)
    def _():
        def _block(b, carry):
            base = s * EPT + b * K
            pltpu.sync_copy(row.at[pl.ds(base, K)], rowbuf)
            pltpu.sync_copy(ew.at[pl.ds(base, K)], ewbuf)
            for jj in range(K // 16):
                ev = ewbuf[pl.ds(jj * 16, 16)]
                for lane in range(16):
                    degsrc.at[jj * 16 + lane][pl.ds(0, 16)] = \
                        _lane_bcast(ev, lane)
            pltpu.sync_copy(degsrc, degacc.at[rowbuf], add=True)
            return carry
        lax.fori_loop(0, NBLK, _block, 0)
    plsc.subcore_barrier()

    @pl.when(c == 0)
    def _():
        pltpu.sync_copy(degacc.at[pl.ds(tbase, STRIPE)],
                        degout.at[pl.ds(tbase, STRIPE)])
    plsc.subcore_barrier()


def _make_deg():
    mesh = plsc.VectorSubcoreMesh(core_axis_name="c", subcore_axis_name="s")
    return pl.kernel(
        _deg_body,
        out_type=jax.ShapeDtypeStruct((NP, 128), _f32),
        mesh=mesh,
        scratch_types=[
            pltpu.VMEM_SHARED((NP, 128), _f32),
            pltpu.VMEM((K,), _i32),
            pltpu.VMEM((K,), _f32),
            pltpu.VMEM((K, 128), _f32),
        ])


_NB = 10
_BN = NP // _NB


def _tc_layer_body(with_pool, x_ref, agg_ref, deg_ref, ws_ref, wa_ref,
                   b_ref, g_ref, be_ref, out_ref, *rest):
    if with_pool:
        pool_ref, y_scr, st_scr = rest
    else:
        y_scr, st_scr = rest
    i = pl.program_id(0)

    @pl.when(i == 0)
    def _init():
        st_scr[...] = jnp.zeros_like(st_scr)

    @pl.when(i < _NB)
    def _phase0():
        deg = deg_ref[:, 0:1]
        r = 1.0 / jnp.maximum(deg, 1.0)
        aggn = agg_ref[...] * r
        dn = (((1,), (1,)), ((), ()))
        y = lax.dot_general(x_ref[...].astype(_bf16),
                            ws_ref[...].astype(_bf16), dn,
                            preferred_element_type=_f32)
        y += lax.dot_general(aggn.astype(_bf16),
                             wa_ref[...].astype(_bf16), dn,
                             preferred_element_type=_f32)
        y += b_ref[...]
        y_scr[pl.ds(i * _BN, _BN), :] = y
        gr = i * _BN + lax.broadcasted_iota(_i32, (_BN, 1), 0)
        m = (gr < N).astype(_f32)
        ym = y * m
        st_scr[0:1, :] += jnp.sum(ym, axis=0, keepdims=True)
        st_scr[1:2, :] += jnp.sum(ym * y, axis=0, keepdims=True)

    @pl.when(i >= _NB)
    def _phase1():
        j = i - _NB
        mu = st_scr[0:1, :] * (1.0 / N)
        var = st_scr[1:2, :] * (1.0 / N) - mu * mu
        inv = lax.rsqrt(var + EPS)
        y = y_scr[pl.ds(j * _BN, _BN), :]
        h = (y - mu) * inv * g_ref[...] + be_ref[...]
        h = jnp.maximum(h, 0.0)
        out_ref[...] = h
        if with_pool:
            gr = j * _BN + lax.broadcasted_iota(_i32, (_BN, 1), 0)
            m = (gr < N).astype(_f32)
            st_scr[2:3, :] += jnp.sum(h * m, axis=0, keepdims=True)

            @pl.when(i == 2 * _NB - 1)
            def _():
                pool_ref[...] = st_scr[2:3, :]


def _make_tc_layer(din, with_pool):
    in_specs = [
        pl.BlockSpec((_BN, din), lambda i: (jnp.where(i < _NB, i, 0), 0)),
        pl.BlockSpec((_BN, din), lambda i: (jnp.where(i < _NB, i, 0), 0)),
        pl.BlockSpec((_BN, 128), lambda i: (jnp.where(i < _NB, i, 0), 0)),
        pl.BlockSpec((H, din), lambda i: (0, 0)),
        pl.BlockSpec((H, din), lambda i: (0, 0)),
        pl.BlockSpec((1, H), lambda i: (0, 0)),
        pl.BlockSpec((1, H), lambda i: (0, 0)),
        pl.BlockSpec((1, H), lambda i: (0, 0)),
    ]
    out_specs = pl.BlockSpec((_BN, H), lambda i: (jnp.where(i < _NB, 0, i - _NB), 0))
    out_shape = jax.ShapeDtypeStruct((NP, H), _f32)
    if with_pool:
        out_specs = [out_specs, pl.BlockSpec((1, H), lambda i: (0, 0))]
        out_shape = [out_shape, jax.ShapeDtypeStruct((1, H), _f32)]
    return pl.pallas_call(
        functools.partial(_tc_layer_body, with_pool),
        grid=(2 * _NB,),
        in_specs=in_specs,
        out_specs=out_specs,
        out_shape=out_shape,
        scratch_shapes=[
            pltpu.VMEM((NP, H), _f32),
            pltpu.VMEM((8, H), _f32),
        ],
    )


def _head_body(pool_ref, w1_ref, b1_ref, w2_ref, b2_ref, o_ref):
    dn = (((1,), (1,)), ((), ()))
    pooled = pool_ref[...] * (1.0 / N)
    z = lax.dot_general(pooled.astype(_bf16), w1_ref[...].astype(_bf16), dn,
                        preferred_element_type=_f32)
    z = jnp.maximum(z + b1_ref[...], 0.0)
    zw = (z.astype(_bf16).astype(_f32)
          * w2_ref[...].astype(_bf16).astype(_f32))
    o_ref[0, 0] = jnp.sum(zw) + b2_ref[0, 0]


_head = pl.pallas_call(
    _head_body,
    out_specs=pl.BlockSpec(memory_space=pltpu.SMEM),
    out_shape=jax.ShapeDtypeStruct((1, 1), _f32),
)


def kernel(x, edge_index, edge_attr,
           W0, b0, g0, be0, W1, b1, g1, be1,
           W2, b2, g2, be2, W3, b3, g3, be3,
           fc1_W, fc1_b, fc2_W, fc2_b):
    row = edge_index[0]
    col = edge_index[1]
    ew = edge_attr.reshape(E)
    Ws = [W0, W1, W2, W3]
    bs = [b0, b1, b2, b3]
    gs = [g0, g1, g2, g3]
    bes = [be0, be1, be2, be3]

    h = jnp.zeros((NP, D_IN), _f32).at[:N].set(x)
    din = D_IN
    deg128 = _make_deg()(row, ew)
    pooled = None
    for i in range(4):
        x2d = h.reshape(NP * (din // 128), 128)
        aggsum = _make_sc_agg(din)(x2d, col, row, ew)
        Wself = Ws[i][:, :din]
        Wagg = Ws[i][:, din:]
        b2d = bs[i].reshape(1, H)
        g2d = gs[i].reshape(1, H)
        be2d = bes[i].reshape(1, H)
        layer = _make_tc_layer(din, i == 3)
        if i == 3:
            h, pooled = layer(h, aggsum, deg128, Wself, Wagg, b2d, g2d, be2d)
        else:
            h = layer(h, aggsum, deg128, Wself, Wagg, b2d, g2d, be2d)
        din = H

    out = _head(pooled, fc1_W, fc1_b.reshape(1, -1),
                fc2_W, fc2_b.reshape(1, 1))
    return out


# speedup vs baseline: 6.7527x; 3.0404x over previous
import functools

import jax
import jax.numpy as jnp
from jax import lax
from jax.experimental import pallas as pl
from jax.experimental.pallas import tpu as pltpu
from jax.experimental.pallas import tpu_sc as plsc

N = 10000
NP = 10240
E = 160000
D_IN = 256
H = 512
NSC = 2
NTEC = 16
EPT = E // NTEC
K = 40
TPB = EPT // K
U = 5
G = TPB // U
STRIPE = NP // NTEC
KD = 40
EPW = E // (NSC * NTEC)
TPBD = EPW // KD
EPS = 1e-5

_f32 = jnp.float32
_i32 = jnp.int32
_bf16 = jnp.bfloat16

_GDN = lax.GatherDimensionNumbers(
    offset_dims=(), collapsed_slice_dims=(0,), start_index_map=(0,))


def _lane_bcast(vec, lane):
    idx = jnp.full((16, 1), lane, _i32)
    return lax.gather(vec, idx, dimension_numbers=_GDN, slice_sizes=(1,),
                      mode=lax.GatherScatterMode.PROMISE_IN_BOUNDS)


def _sc_agg_body(nq, *refs):
    xchunks = refs[:nq]
    col, row, ew = refs[nq:nq + 3]
    outchunks = refs[nq + 3:2 * nq + 3]
    (acc, colb, rowb, ewb, rowc, rows5) = refs[2 * nq + 3:2 * nq + 9]
    gsems = refs[2 * nq + 9:2 * nq + 9 + U]
    ssems = refs[2 * nq + 9 + U:2 * nq + 9 + 2 * U]
    isems = refs[2 * nq + 9 + 2 * U:2 * nq + 9 + 3 * U]
    qper = nq // NSC
    c = lax.axis_index("c")
    s = lax.axis_index("s")
    tbase = s * STRIPE
    zero16 = jnp.zeros((16,), _f32)

    def _zero_buf0():
        def _z(i, carry):
            rr = rows5.at[0].at[i]
            for j in range(8):
                rr[pl.ds(j * 16, 16)] = zero16
            return carry
        lax.fori_loop(0, K, _z, 0)

    def _issue_idx(b, slot):
        base = s * EPT + b * K
        pltpu.async_copy(col.at[pl.ds(base, K)], colb.at[slot], isems[slot])
        pltpu.async_copy(row.at[pl.ds(base, K)], rowb.at[slot], isems[slot])
        pltpu.async_copy(ew.at[pl.ds(base, K)], ewb.at[slot], isems[slot])

    def _wait_idx(slot):
        pltpu.make_async_copy(col.at[pl.ds(0, K)], colb.at[slot],
                              isems[slot]).wait()
        pltpu.make_async_copy(row.at[pl.ds(0, K)], rowb.at[slot],
                              isems[slot]).wait()
        pltpu.make_async_copy(ew.at[pl.ds(0, K)], ewb.at[slot],
                              isems[slot]).wait()

    def _issue_gather(qi, slot):
        for cval in range(NSC):
            @pl.when(c == cval)
            def _():
                src = xchunks[cval * qper + qi]
                pltpu.async_copy(src.at[colb.at[slot]], rows5.at[slot],
                                 gsems[slot])

    def _wait_gather(slot):
        pltpu.make_async_copy(xchunks[0].at[pl.ds(0, K)], rows5.at[slot],
                              gsems[slot]).wait()

    def _issue_scatter(slot):
        pltpu.async_copy(rows5.at[slot], acc.at[rowc.at[slot]], ssems[slot],
                         add=True)

    def _wait_scatter(slot):
        pltpu.make_async_copy(rows5.at[slot], acc.at[pl.ds(0, K)],
                              ssems[slot]).wait()

    for qi in range(qper):
        _zero_buf0()
        for z in range(STRIPE // K):
            pltpu.sync_copy(rows5.at[0], acc.at[pl.ds(tbase + z * K, K)])
        plsc.subcore_barrier()

        _issue_idx(0, 0)
        _issue_idx(1, 1)
        _issue_idx(2, 2)
        _wait_idx(0)
        _issue_gather(qi, 0)
        _wait_idx(1)
        _issue_gather(qi, 1)

        def _group(g, carry):
            for u in range(U):
                b = g * U + u
                u2 = (u + 2) % U
                u3 = (u + 3) % U

                @pl.when(b + 3 < TPB)
                def _():
                    _issue_idx(b + 3, u3)

                @pl.when(b + 2 < TPB)
                def _():
                    @pl.when(b >= 3)
                    def _():
                        _wait_scatter(u2)
                    _wait_idx(u2)
                    _issue_gather(qi, u2)

                _wait_gather(u)

                rc = rowc.at[u]
                rb = rowb.at[u]
                rc[pl.ds(0, 16)] = rb[pl.ds(0, 16)]
                rc[pl.ds(16, 16)] = rb[pl.ds(16, 16)]
                rc[pl.ds(24, 16)] = rb[pl.ds(24, 16)]

                eb = ewb.at[u]
                ru = rows5.at[u]

                def _scale(grp, inner):
                    ew16 = eb[pl.ds(grp * 16, 16)]
                    for lane in range(16):
                        w = _lane_bcast(ew16, lane)
                        rr = ru.at[grp * 16 + lane]
                        for j in range(8):
                            sl = pl.ds(j * 16, 16)
                            rr[sl] = rr[sl] * w
                    return inner
                lax.fori_loop(0, K // 16, _scale, 0)
                ewt = eb[pl.ds(24, 16)]
                for lane in range(8, 16):
                    w = _lane_bcast(ewt, lane)
                    rr = ru.at[24 + lane]
                    for j in range(8):
                        sl = pl.ds(j * 16, 16)
                        rr[sl] = rr[sl] * w

                _issue_scatter(u)
            return carry
        lax.fori_loop(0, G, _group, 0)
        for u in range(U):
            _wait_scatter(u)
        plsc.subcore_barrier()

        for cval in range(NSC):
            @pl.when(c == cval)
            def _():
                pltpu.sync_copy(
                    acc.at[pl.ds(tbase, STRIPE)],
                    outchunks[cval * qper + qi].at[pl.ds(tbase, STRIPE)])
        plsc.subcore_barrier()


def _make_sc_agg(nq):
    mesh = plsc.VectorSubcoreMesh(core_axis_name="c", subcore_axis_name="s")
    out_type = tuple(jax.ShapeDtypeStruct((NP, 128), _f32)
                     for _ in range(nq))
    scratch = [
        pltpu.VMEM_SHARED((NP, 128), _f32),
        pltpu.VMEM((U, K), _i32),
        pltpu.VMEM((U, K), _i32),
        pltpu.VMEM((U, K), _f32),
        pltpu.VMEM((U, K), _i32),
        pltpu.VMEM((U, K, 128), _f32),
    ]
    scratch += [pltpu.SemaphoreType.DMA] * (3 * U)

    def body(*refs):
        _sc_agg_body(nq, *refs)

    return pl.kernel(body, out_type=out_type, mesh=mesh,
                     scratch_types=scratch)


def _deg_body(row2d, ew2d, degout, degacc, rowv, ewv, degsrc):
    c = lax.axis_index("c")
    s = lax.axis_index("s")
    tbase = s * STRIPE
    w = s * NSC + c
    zero16 = jnp.zeros((16,), _f32)

    pltpu.sync_copy(row2d.at[w], rowv)
    pltpu.sync_copy(ew2d.at[w], ewv)

    def _zdeg(i, carry):
        dd = degsrc.at[i]
        for j in range(8):
            dd[pl.ds(j * 16, 16)] = zero16
        return carry
    lax.fori_loop(0, KD, _zdeg, 0)
    for z in range(STRIPE // KD):
        pltpu.sync_copy(degsrc, degacc.at[pl.ds(tbase + z * KD, KD)])
    plsc.subcore_barrier()

    def _block(b, carry):
        er = ewv.at[b]
        for jj in range(KD // 16):
            ew16 = er[pl.ds(jj * 16, 16)]
            for lane in range(16):
                degsrc.at[jj * 16 + lane][pl.ds(0, 16)] = \
                    _lane_bcast(ew16, lane)
        ew8 = er[pl.ds(24, 16)]
        for lane in range(8, 16):
            degsrc.at[jj * 16 + lane + 8][pl.ds(0, 16)] = \
                _lane_bcast(ew8, lane)
        pltpu.sync_copy(degsrc, degacc.at[rowv.at[b]], add=True)
        return carry
    lax.fori_loop(0, TPBD, _block, 0)
    plsc.subcore_barrier()

    for cval in range(NSC):
        @pl.when(c == cval)
        def _():
            pltpu.sync_copy(degacc.at[pl.ds(tbase, STRIPE)],
                            degout.at[pl.ds(tbase, STRIPE),
                                      pl.ds(cval * 128, 128)])
    plsc.subcore_barrier()


def _make_deg():
    mesh = plsc.VectorSubcoreMesh(core_axis_name="c", subcore_axis_name="s")
    return pl.kernel(
        _deg_body,
        out_type=jax.ShapeDtypeStruct((NP, 256), _f32),
        mesh=mesh,
        scratch_types=[
            pltpu.VMEM_SHARED((NP, 128), _f32),
            pltpu.VMEM((TPBD, KD), _i32),
            pltpu.VMEM((TPBD, KD), _f32),
            pltpu.VMEM((KD, 128), _f32),
        ])


_NB = 10
_BN = NP // _NB


def _tc_layer_body(nq, with_pool, *refs):
    xrefs = refs[:nq]
    aggrefs = refs[nq:2 * nq]
    deg_ref, ws_ref, wa_ref, b_ref, g_ref, be_ref = refs[2 * nq:2 * nq + 6]
    nq_out = H // 128
    outrefs = refs[2 * nq + 6:2 * nq + 6 + nq_out]
    rest = refs[2 * nq + 6 + nq_out:]
    if with_pool:
        pool_ref, y_scr, st_scr = rest
    else:
        y_scr, st_scr = rest
    i = pl.program_id(0)

    @pl.when(i == 0)
    def _init():
        st_scr[...] = jnp.zeros_like(st_scr)

    @pl.when(i < _NB)
    def _phase0():
        deg = deg_ref[:, 0:1] + deg_ref[:, 128:129]
        r = 1.0 / jnp.maximum(deg, 1.0)
        x = jnp.concatenate([xr[...] for xr in xrefs], axis=-1)
        agg = jnp.concatenate([ar[...] for ar in aggrefs], axis=-1)
        aggn = agg * r
        dn = (((1,), (1,)), ((), ()))
        y = lax.dot_general(x.astype(_bf16), ws_ref[...].astype(_bf16), dn,
                            preferred_element_type=_f32)
        y += lax.dot_general(aggn.astype(_bf16), wa_ref[...].astype(_bf16),
                             dn, preferred_element_type=_f32)
        y += b_ref[...]
        y_scr[pl.ds(i * _BN, _BN), :] = y
        gr = i * _BN + lax.broadcasted_iota(_i32, (_BN, 1), 0)
        m = (gr < N).astype(_f32)
        ym = y * m
        st_scr[0:1, :] += jnp.sum(ym, axis=0, keepdims=True)
        st_scr[1:2, :] += jnp.sum(ym * y, axis=0, keepdims=True)

    @pl.when(i >= _NB)
    def _phase1():
        j = i - _NB
        mu = st_scr[0:1, :] * (1.0 / N)
        var = st_scr[1:2, :] * (1.0 / N) - mu * mu
        inv = lax.rsqrt(var + EPS)
        y = y_scr[pl.ds(j * _BN, _BN), :]
        h = (y - mu) * inv * g_ref[...] + be_ref[...]
        h = jnp.maximum(h, 0.0)
        for q in range(nq_out):
            outrefs[q][...] = h[:, q * 128:(q + 1) * 128]
        if with_pool:
            gr = j * _BN + lax.broadcasted_iota(_i32, (_BN, 1), 0)
            m = (gr < N).astype(_f32)
            st_scr[2:3, :] += jnp.sum(h * m, axis=0, keepdims=True)

            @pl.when(i == 2 * _NB - 1)
            def _():
                pool_ref[...] = st_scr[2:3, :]


def _make_tc_layer(nq_in, with_pool):
    din = nq_in * 128
    nq_out = H // 128
    blk = lambda i: (jnp.where(i < _NB, i, 0), 0)
    in_specs = (
        [pl.BlockSpec((_BN, 128), blk)] * nq_in
        + [pl.BlockSpec((_BN, 128), blk)] * nq_in
        + [pl.BlockSpec((_BN, 256), blk)]
        + [pl.BlockSpec((H, din), lambda i: (0, 0))] * 2
        + [pl.BlockSpec((1, H), lambda i: (0, 0))] * 3
    )
    oblk = lambda i: (jnp.where(i < _NB, 0, i - _NB), 0)
    out_specs = [pl.BlockSpec((_BN, 128), oblk)] * nq_out
    out_shape = [jax.ShapeDtypeStruct((NP, 128), _f32)] * nq_out
    if with_pool:
        out_specs.append(pl.BlockSpec((1, H), lambda i: (0, 0)))
        out_shape.append(jax.ShapeDtypeStruct((1, H), _f32))
    return pl.pallas_call(
        functools.partial(_tc_layer_body, nq_in, with_pool),
        grid=(2 * _NB,),
        in_specs=in_specs,
        out_specs=out_specs,
        out_shape=out_shape,
        scratch_shapes=[
            pltpu.VMEM((NP, H), _f32),
            pltpu.VMEM((8, H), _f32),
        ],
    )


def _head_body(pool_ref, w1_ref, b1_ref, w2_ref, b2_ref, o_ref):
    dn = (((1,), (1,)), ((), ()))
    pooled = pool_ref[...] * (1.0 / N)
    z = lax.dot_general(pooled.astype(_bf16), w1_ref[...].astype(_bf16), dn,
                        preferred_element_type=_f32)
    z = jnp.maximum(z + b1_ref[...], 0.0)
    zw = (z.astype(_bf16).astype(_f32)
          * w2_ref[...].astype(_bf16).astype(_f32))
    o_ref[0, 0] = jnp.sum(zw) + b2_ref[0, 0]


_head = pl.pallas_call(
    _head_body,
    out_specs=pl.BlockSpec(memory_space=pltpu.SMEM),
    out_shape=jax.ShapeDtypeStruct((1, 1), _f32),
)


def kernel(x, edge_index, edge_attr,
           W0, b0, g0, be0, W1, b1, g1, be1,
           W2, b2, g2, be2, W3, b3, g3, be3,
           fc1_W, fc1_b, fc2_W, fc2_b):
    row = edge_index[0]
    col = edge_index[1]
    ew = edge_attr.reshape(E)
    row2dd = row.reshape(NSC * NTEC, TPBD, KD)
    ew2dd = ew.reshape(NSC * NTEC, TPBD, KD)
    Ws = [W0, W1, W2, W3]
    bs = [b0, b1, b2, b3]
    gs = [g0, g1, g2, g3]
    bes = [be0, be1, be2, be3]

    xp = jnp.zeros((NP, D_IN), _f32).at[:N].set(x)
    hc = [xp[:, 0:128], xp[:, 128:256]]
    deg2 = _make_deg()(row2dd, ew2dd)
    pooled = None
    for i in range(4):
        nq = len(hc)
        aggc = _make_sc_agg(nq)(*hc, col, row, ew)
        din = nq * 128
        Wself = Ws[i][:, :din]
        Wagg = Ws[i][:, din:]
        b2d = bs[i].reshape(1, H)
        g2d = gs[i].reshape(1, H)
        be2d = bes[i].reshape(1, H)
        layer = _make_tc_layer(nq, i == 3)
        outs = layer(*hc, *aggc, deg2, Wself, Wagg, b2d, g2d, be2d)
        if i == 3:
            hc, pooled = outs[:-1], outs[-1]
        else:
            hc = outs

    out = _head(pooled, fc1_W, fc1_b.reshape(1, -1),
                fc2_W, fc2_b.reshape(1, 1))
    return out


# speedup vs baseline: 7.1549x; 1.0596x over previous
import functools

import jax
import jax.numpy as jnp
from jax import lax
from jax.experimental import pallas as pl
from jax.experimental.pallas import tpu as pltpu
from jax.experimental.pallas import tpu_sc as plsc

N = 10000
NP = 10240
E = 160000
D_IN = 256
H = 512
NSC = 2
NTEC = 16
EPT = E // NTEC
K = 80
TPB = EPT // K
U = 4
G = TPB // U
STRIPE = NP // NTEC
KD = 40
EPW = E // (NSC * NTEC)
TPBD = EPW // KD
EPS = 1e-5

_f32 = jnp.float32
_i32 = jnp.int32
_bf16 = jnp.bfloat16

_GDN = lax.GatherDimensionNumbers(
    offset_dims=(), collapsed_slice_dims=(0,), start_index_map=(0,))


def _lane_bcast(vec, lane):
    idx = jnp.full((16, 1), lane, _i32)
    return lax.gather(vec, idx, dimension_numbers=_GDN, slice_sizes=(1,),
                      mode=lax.GatherScatterMode.PROMISE_IN_BOUNDS)


def _sc_agg_body(nq, *refs):
    xchunks = refs[:nq]
    col, row, ew = refs[nq:nq + 3]
    outchunks = refs[nq + 3:2 * nq + 3]
    (acc, colb, rowb, ewb, rowc, rows5) = refs[2 * nq + 3:2 * nq + 9]
    gsems = refs[2 * nq + 9:2 * nq + 9 + U]
    ssems = refs[2 * nq + 9 + U:2 * nq + 9 + 2 * U]
    isems = refs[2 * nq + 9 + 2 * U:2 * nq + 9 + 3 * U]
    qper = nq // NSC
    c = lax.axis_index("c")
    s = lax.axis_index("s")
    tbase = s * STRIPE
    zero16 = jnp.zeros((16,), _f32)

    def _zero_buf0():
        def _z(i, carry):
            rr = rows5.at[0].at[i]
            for j in range(8):
                rr[pl.ds(j * 16, 16)] = zero16
            return carry
        lax.fori_loop(0, K, _z, 0)

    def _issue_idx(b, slot):
        base = s * EPT + b * K
        pltpu.async_copy(col.at[pl.ds(base, K)], colb.at[slot], isems[slot])
        pltpu.async_copy(row.at[pl.ds(base, K)], rowb.at[slot], isems[slot])
        pltpu.async_copy(ew.at[pl.ds(base, K)], ewb.at[slot], isems[slot])

    def _wait_idx(slot):
        pltpu.make_async_copy(col.at[pl.ds(0, K)], colb.at[slot],
                              isems[slot]).wait()
        pltpu.make_async_copy(row.at[pl.ds(0, K)], rowb.at[slot],
                              isems[slot]).wait()
        pltpu.make_async_copy(ew.at[pl.ds(0, K)], ewb.at[slot],
                              isems[slot]).wait()

    def _issue_gather(qi, slot):
        for cval in range(NSC):
            @pl.when(c == cval)
            def _():
                src = xchunks[cval * qper + qi]
                pltpu.async_copy(src.at[colb.at[slot]], rows5.at[slot],
                                 gsems[slot])

    def _wait_gather(slot):
        pltpu.make_async_copy(xchunks[0].at[pl.ds(0, K)], rows5.at[slot],
                              gsems[slot]).wait()

    def _issue_scatter(slot):
        pltpu.async_copy(rows5.at[slot], acc.at[rowc.at[slot]], ssems[slot],
                         add=True)

    def _wait_scatter(slot):
        pltpu.make_async_copy(rows5.at[slot], acc.at[pl.ds(0, K)],
                              ssems[slot]).wait()

    def _process_block(u):
        _wait_gather(u)
        rc = rowc.at[u]
        rb = rowb.at[u]
        for t in range(K // 16):
            rc[pl.ds(t * 16, 16)] = rb[pl.ds(t * 16, 16)]
        eb = ewb.at[u]
        ru = rows5.at[u]

        def _scale(grp, inner):
            ew16 = eb[pl.ds(grp * 16, 16)]
            for lane in range(16):
                w = _lane_bcast(ew16, lane)
                rr = ru.at[grp * 16 + lane]
                for j in range(8):
                    sl = pl.ds(j * 16, 16)
                    rr[sl] = rr[sl] * w
            return inner
        lax.fori_loop(0, K // 16, _scale, 0)
        _issue_scatter(u)

    for qi in range(qper):
        _zero_buf0()
        for z in range(STRIPE // K):
            pltpu.sync_copy(rows5.at[0], acc.at[pl.ds(tbase + z * K, K)])
        plsc.subcore_barrier()

        _issue_idx(0, 0)
        _issue_idx(1, 1)
        _issue_idx(2, 2)
        _wait_idx(0)
        _issue_gather(qi, 0)
        _wait_idx(1)
        _issue_gather(qi, 1)

        def _group(g, carry):
            for u in range(U):
                b = g * U + u
                u2 = (u + 2) % U
                u3 = (u + 3) % U

                @pl.when(b + 3 < TPB)
                def _():
                    _issue_idx(b + 3, u3)

                @pl.when(b + 2 < TPB)
                def _():
                    @pl.when(b >= U - 2)
                    def _():
                        _wait_scatter(u2)
                    _wait_idx(u2)
                    _issue_gather(qi, u2)

                _process_block(u)
            return carry
        lax.fori_loop(0, G, _group, 0)

        _process_block((TPB - 1) % U)
        for u in range(U):
            _wait_scatter(u)
        plsc.subcore_barrier()

        for cval in range(NSC):
            @pl.when(c == cval)
            def _():
                pltpu.sync_copy(
                    acc.at[pl.ds(tbase, STRIPE)],
                    outchunks[cval * qper + qi].at[pl.ds(tbase, STRIPE)])
        plsc.subcore_barrier()


def _make_sc_agg(nq):
    mesh = plsc.VectorSubcoreMesh(core_axis_name="c", subcore_axis_name="s")
    out_type = tuple(jax.ShapeDtypeStruct((NP, 128), _f32)
                     for _ in range(nq))
    scratch = [
        pltpu.VMEM_SHARED((NP, 128), _f32),
        pltpu.VMEM((U, K), _i32),
        pltpu.VMEM((U, K), _i32),
        pltpu.VMEM((U, K), _f32),
        pltpu.VMEM((U, K), _i32),
        pltpu.VMEM((U, K, 128), _f32),
    ]
    scratch += [pltpu.SemaphoreType.DMA] * (3 * U)

    def body(*refs):
        _sc_agg_body(nq, *refs)

    return pl.kernel(body, out_type=out_type, mesh=mesh,
                     scratch_types=scratch)


def _deg_body(row2d, ew2d, degout, degacc, rowv, ewv, degsrc):
    c = lax.axis_index("c")
    s = lax.axis_index("s")
    tbase = s * STRIPE
    w = s * NSC + c
    zero16 = jnp.zeros((16,), _f32)

    pltpu.sync_copy(row2d.at[w], rowv)
    pltpu.sync_copy(ew2d.at[w], ewv)

    def _zdeg(i, carry):
        dd = degsrc.at[i]
        for j in range(8):
            dd[pl.ds(j * 16, 16)] = zero16
        return carry
    lax.fori_loop(0, KD, _zdeg, 0)
    for z in range(STRIPE // KD):
        pltpu.sync_copy(degsrc, degacc.at[pl.ds(tbase + z * KD, KD)])
    plsc.subcore_barrier()

    def _block(b, carry):
        er = ewv.at[b]
        for jj in range(KD // 16):
            ew16 = er[pl.ds(jj * 16, 16)]
            for lane in range(16):
                degsrc.at[jj * 16 + lane][pl.ds(0, 16)] = \
                    _lane_bcast(ew16, lane)
        ew8 = er[pl.ds(24, 16)]
        for lane in range(8, 16):
            degsrc.at[jj * 16 + lane + 8][pl.ds(0, 16)] = \
                _lane_bcast(ew8, lane)
        pltpu.sync_copy(degsrc, degacc.at[rowv.at[b]], add=True)
        return carry
    lax.fori_loop(0, TPBD, _block, 0)
    plsc.subcore_barrier()

    for cval in range(NSC):
        @pl.when(c == cval)
        def _():
            pltpu.sync_copy(degacc.at[pl.ds(tbase, STRIPE)],
                            degout.at[pl.ds(tbase, STRIPE),
                                      pl.ds(cval * 128, 128)])
    plsc.subcore_barrier()


def _make_deg():
    mesh = plsc.VectorSubcoreMesh(core_axis_name="c", subcore_axis_name="s")
    return pl.kernel(
        _deg_body,
        out_type=jax.ShapeDtypeStruct((NP, 256), _f32),
        mesh=mesh,
        scratch_types=[
            pltpu.VMEM_SHARED((NP, 128), _f32),
            pltpu.VMEM((TPBD, KD), _i32),
            pltpu.VMEM((TPBD, KD), _f32),
            pltpu.VMEM((KD, 128), _f32),
        ])


_NB = 10
_BN = NP // _NB


def _tc_layer_body(nq, with_pool, *refs):
    xrefs = refs[:nq]
    aggrefs = refs[nq:2 * nq]
    deg_ref, ws_ref, wa_ref, b_ref, g_ref, be_ref = refs[2 * nq:2 * nq + 6]
    nq_out = H // 128
    outrefs = refs[2 * nq + 6:2 * nq + 6 + nq_out]
    rest = refs[2 * nq + 6 + nq_out:]
    if with_pool:
        pool_ref, y_scr, st_scr = rest
    else:
        y_scr, st_scr = rest
    i = pl.program_id(0)

    @pl.when(i == 0)
    def _init():
        st_scr[...] = jnp.zeros_like(st_scr)

    @pl.when(i < _NB)
    def _phase0():
        deg = deg_ref[:, 0:1] + deg_ref[:, 128:129]
        r = 1.0 / jnp.maximum(deg, 1.0)
        x = jnp.concatenate([xr[...] for xr in xrefs], axis=-1)
        agg = jnp.concatenate([ar[...] for ar in aggrefs], axis=-1)
        aggn = agg * r
        dn = (((1,), (1,)), ((), ()))
        y = lax.dot_general(x.astype(_bf16), ws_ref[...].astype(_bf16), dn,
                            preferred_element_type=_f32)
        y += lax.dot_general(aggn.astype(_bf16), wa_ref[...].astype(_bf16),
                             dn, preferred_element_type=_f32)
        y += b_ref[...]
        y_scr[pl.ds(i * _BN, _BN), :] = y
        gr = i * _BN + lax.broadcasted_iota(_i32, (_BN, 1), 0)
        m = (gr < N).astype(_f32)
        ym = y * m
        st_scr[0:1, :] += jnp.sum(ym, axis=0, keepdims=True)
        st_scr[1:2, :] += jnp.sum(ym * y, axis=0, keepdims=True)

    @pl.when(i >= _NB)
    def _phase1():
        j = i - _NB
        mu = st_scr[0:1, :] * (1.0 / N)
        var = st_scr[1:2, :] * (1.0 / N) - mu * mu
        inv = lax.rsqrt(var + EPS)
        y = y_scr[pl.ds(j * _BN, _BN), :]
        h = (y - mu) * inv * g_ref[...] + be_ref[...]
        h = jnp.maximum(h, 0.0)
        for q in range(nq_out):
            outrefs[q][...] = h[:, q * 128:(q + 1) * 128]
        if with_pool:
            gr = j * _BN + lax.broadcasted_iota(_i32, (_BN, 1), 0)
            m = (gr < N).astype(_f32)
            st_scr[2:3, :] += jnp.sum(h * m, axis=0, keepdims=True)

            @pl.when(i == 2 * _NB - 1)
            def _():
                pool_ref[...] = st_scr[2:3, :]


def _make_tc_layer(nq_in, with_pool):
    din = nq_in * 128
    nq_out = H // 128
    blk = lambda i: (jnp.where(i < _NB, i, 0), 0)
    in_specs = (
        [pl.BlockSpec((_BN, 128), blk)] * nq_in
        + [pl.BlockSpec((_BN, 128), blk)] * nq_in
        + [pl.BlockSpec((_BN, 256), blk)]
        + [pl.BlockSpec((H, din), lambda i: (0, 0))] * 2
        + [pl.BlockSpec((1, H), lambda i: (0, 0))] * 3
    )
    oblk = lambda i: (jnp.where(i < _NB, 0, i - _NB), 0)
    out_specs = [pl.BlockSpec((_BN, 128), oblk)] * nq_out
    out_shape = [jax.ShapeDtypeStruct((NP, 128), _f32)] * nq_out
    if with_pool:
        out_specs.append(pl.BlockSpec((1, H), lambda i: (0, 0)))
        out_shape.append(jax.ShapeDtypeStruct((1, H), _f32))
    return pl.pallas_call(
        functools.partial(_tc_layer_body, nq_in, with_pool),
        grid=(2 * _NB,),
        in_specs=in_specs,
        out_specs=out_specs,
        out_shape=out_shape,
        scratch_shapes=[
            pltpu.VMEM((NP, H), _f32),
            pltpu.VMEM((8, H), _f32),
        ],
    )


def _head_body(pool_ref, w1_ref, b1_ref, w2_ref, b2_ref, o_ref):
    dn = (((1,), (1,)), ((), ()))
    pooled = pool_ref[...] * (1.0 / N)
    z = lax.dot_general(pooled.astype(_bf16), w1_ref[...].astype(_bf16), dn,
                        preferred_element_type=_f32)
    z = jnp.maximum(z + b1_ref[...], 0.0)
    zw = (z.astype(_bf16).astype(_f32)
          * w2_ref[...].astype(_bf16).astype(_f32))
    o_ref[0, 0] = jnp.sum(zw) + b2_ref[0, 0]


_head = pl.pallas_call(
    _head_body,
    out_specs=pl.BlockSpec(memory_space=pltpu.SMEM),
    out_shape=jax.ShapeDtypeStruct((1, 1), _f32),
)


def kernel(x, edge_index, edge_attr,
           W0, b0, g0, be0, W1, b1, g1, be1,
           W2, b2, g2, be2, W3, b3, g3, be3,
           fc1_W, fc1_b, fc2_W, fc2_b):
    row = edge_index[0]
    col = edge_index[1]
    ew = edge_attr.reshape(E)
    row2dd = row.reshape(NSC * NTEC, TPBD, KD)
    ew2dd = ew.reshape(NSC * NTEC, TPBD, KD)
    Ws = [W0, W1, W2, W3]
    bs = [b0, b1, b2, b3]
    gs = [g0, g1, g2, g3]
    bes = [be0, be1, be2, be3]

    xp = jnp.zeros((NP, D_IN), _f32).at[:N].set(x)
    hc = [xp[:, 0:128], xp[:, 128:256]]
    deg2 = _make_deg()(row2dd, ew2dd)
    pooled = None
    for i in range(4):
        nq = len(hc)
        aggc = _make_sc_agg(nq)(*hc, col, row, ew)
        din = nq * 128
        Wself = Ws[i][:, :din]
        Wagg = Ws[i][:, din:]
        b2d = bs[i].reshape(1, H)
        g2d = gs[i].reshape(1, H)
        be2d = bes[i].reshape(1, H)
        layer = _make_tc_layer(nq, i == 3)
        outs = layer(*hc, *aggc, deg2, Wself, Wagg, b2d, g2d, be2d)
        if i == 3:
            hc, pooled = outs[:-1], outs[-1]
        else:
            hc = outs

    out = _head(pooled, fc1_W, fc1_b.reshape(1, -1),
                fc2_W, fc2_b.reshape(1, 1))
    return out
